# baseline (speedup 1.0000x reference)
import functools

import jax
import jax.numpy as jnp
from jax.experimental import pallas as pl
from jax.experimental.pallas import tpu as pltpu

N_PAD = 50176
BN = 1024


def _h_body(x_ref, a_ref, w0l_ref, b0l_ref, w0r_ref, h_ref):
    x = x_ref[...]
    a = a_ref[...]
    pre = a * w0l_ref[...] + x * w0r_ref[...] + b0l_ref[...]
    h_ref[...] = jnp.maximum(pre, 0.0) + x


def _make_h(x_p, a_p, W0l, b0l, W0r):
    grid = (N_PAD // BN,)
    return pl.pallas_call(
        _h_body,
        grid=grid,
        in_specs=[
            pl.BlockSpec((BN, 1), lambda i: (i, 0)),
            pl.BlockSpec((BN, 1), lambda i: (i, 0)),
            pl.BlockSpec((1, 128), lambda i: (0, 0)),
            pl.BlockSpec((1, 128), lambda i: (0, 0)),
            pl.BlockSpec((1, 128), lambda i: (0, 0)),
        ],
        out_specs=pl.BlockSpec((BN, 128), lambda i: (i, 0)),
        out_shape=jax.ShapeDtypeStruct((N_PAD, 128), jnp.float32),
    )(x_p, a_p, W0l, b0l.reshape(1, 128), W0r)


def _tail_body(agg2_ref, invd_ref, h_ref, batch_ref,
               w1l_ref, b1l_ref, w1r_ref,
               fc1w_ref, fc1b_ref, fc2w_ref, fc2b_ref,
               out_ref, psum_ref, pcnt_ref):
    i = pl.program_id(0)
    nb = pl.num_programs(0)

    @pl.when(i == 0)
    def _init():
        psum_ref[...] = jnp.zeros_like(psum_ref)
        pcnt_ref[...] = jnp.zeros_like(pcnt_ref)

    h = h_ref[...]
    aggd = agg2_ref[...] * invd_ref[...]
    pre = (jax.lax.dot(aggd, w1l_ref[...], preferred_element_type=jnp.float32)
           + b1l_ref[...]
           + jax.lax.dot(h, w1r_ref[...], preferred_element_type=jnp.float32))
    h2 = jnp.maximum(pre, 0.0) + h

    b = batch_ref[...]
    gids = jax.lax.broadcasted_iota(jnp.int32, (1, 64), 1)
    onehot = jnp.where(b == gids, 1.0, 0.0)
    psum_ref[...] += jax.lax.dot_general(
        onehot, h2, (((0,), (0,)), ((), ())),
        preferred_element_type=jnp.float32, precision=jax.lax.Precision.HIGHEST)
    pcnt_ref[...] += jax.lax.dot_general(
        onehot, jnp.ones_like(h2), (((0,), (0,)), ((), ())),
        preferred_element_type=jnp.float32, precision=jax.lax.Precision.HIGHEST)

    @pl.when(i == nb - 1)
    def _fin():
        pooled = psum_ref[...] / jnp.maximum(pcnt_ref[...], 1.0)
        z = jnp.maximum(
            jax.lax.dot(pooled, fc1w_ref[...],
                        preferred_element_type=jnp.float32) + fc1b_ref[...],
            0.0)
        out_ref[...] = (jax.lax.dot(z, fc2w_ref[...],
                                    preferred_element_type=jnp.float32)
                        + fc2b_ref[...])


def _tail(agg2_p, invd_p, h_p, batch_p, W1l, b1l, W1r, fc1_W, fc1_b, fc2_W, fc2_b):
    grid = (N_PAD // BN,)
    return pl.pallas_call(
        _tail_body,
        grid=grid,
        in_specs=[
            pl.BlockSpec((BN, 128), lambda i: (i, 0)),
            pl.BlockSpec((BN, 1), lambda i: (i, 0)),
            pl.BlockSpec((BN, 128), lambda i: (i, 0)),
            pl.BlockSpec((BN, 1), lambda i: (i, 0)),
            pl.BlockSpec((128, 128), lambda i: (0, 0)),
            pl.BlockSpec((1, 128), lambda i: (0, 0)),
            pl.BlockSpec((128, 128), lambda i: (0, 0)),
            pl.BlockSpec((128, 64), lambda i: (0, 0)),
            pl.BlockSpec((1, 64), lambda i: (0, 0)),
            pl.BlockSpec((64, 1), lambda i: (0, 0)),
            pl.BlockSpec((1, 1), lambda i: (0, 0)),
        ],
        out_specs=pl.BlockSpec((64, 1), lambda i: (0, 0)),
        out_shape=jax.ShapeDtypeStruct((64, 1), jnp.float32),
        scratch_shapes=[
            pltpu.VMEM((64, 128), jnp.float32),
            pltpu.VMEM((64, 128), jnp.float32),
        ],
    )(agg2_p, invd_p, h_p, batch_p, W1l, b1l.reshape(1, 128), W1r,
      fc1_W, fc1_b.reshape(1, 64), fc2_W, fc2_b.reshape(1, 1))


def kernel(x, edge_index, batch, W0l, b0l, W0r, W1l, b1l, W1r,
           fc1_W, fc1_b, fc2_W, fc2_b):
    n = x.shape[0]
    src = edge_index[0]
    dst = edge_index[1]

    deg = jax.ops.segment_sum(jnp.ones((src.shape[0],), jnp.float32), dst,
                              num_segments=n)
    s1 = jax.ops.segment_sum(x[src, 0], dst, num_segments=n)
    invd = 1.0 / jnp.maximum(deg, 1.0)
    a = s1 * invd

    pad = N_PAD - n
    x_p = jnp.pad(x, ((0, pad), (0, 0)))
    a_p = jnp.pad(a.reshape(n, 1), ((0, pad), (0, 0)))
    invd_p = jnp.pad(invd.reshape(n, 1), ((0, pad), (0, 0)),
                     constant_values=1.0)
    batch_p = jnp.pad(batch.reshape(n, 1), ((0, pad), (0, 0)),
                      constant_values=64)

    h_p = _make_h(x_p, a_p, W0l, b0l, W0r)

    agg2 = jax.ops.segment_sum(h_p[:n][src], dst, num_segments=n)
    agg2_p = jnp.pad(agg2, ((0, pad), (0, 0)))

    return _tail(agg2_p, invd_p, h_p, batch_p, W1l, b1l, W1r,
                 fc1_W, fc1_b, fc2_W, fc2_b)


# speedup vs baseline: 2.7209x; 2.7209x over previous
import functools

import jax
import jax.numpy as jnp
from jax import lax
from jax.experimental import pallas as pl
from jax.experimental.pallas import tpu as pltpu
from jax.experimental.pallas import tpu_sc as plsc

N = 50000
N_PAD = 50176
BN = 1024
E = 800000
E_PAD = 819200
CHUNK = 128
NG = 16
GW = 128 // NG
CHUNKS_TOTAL = E_PAD // CHUNK
CHUNKS_PER_TILE = CHUNKS_TOTAL // 32
CHUNKS_PER_TILE2 = CHUNKS_TOTAL // 16
ROWS_PER_TILE = N_PAD // 16

_sc_mesh = plsc.VectorSubcoreMesh(core_axis_name="c", subcore_axis_name="s")


def _sc1_body(xaug_hbm, src_hbm, dst_hbm, zeros_hbm, out_hbm,
              src_v, dst_v, gbuf, acc_sh, sem):
    c = lax.axis_index("c")
    s = lax.axis_index("s")
    wid = s * 2 + c
    pltpu.sync_copy(src_hbm.at[pl.ds(wid * CHUNKS_PER_TILE, CHUNKS_PER_TILE)],
                    src_v)
    pltpu.sync_copy(dst_hbm.at[pl.ds(wid * CHUNKS_PER_TILE, CHUNKS_PER_TILE)],
                    dst_v)
    pltpu.sync_copy(zeros_hbm.at[pl.ds(0, ROWS_PER_TILE)],
                    acc_sh.at[pl.ds(s * ROWS_PER_TILE, ROWS_PER_TILE)])
    plsc.subcore_barrier()

    def body(j, carry):
        pltpu.async_copy(xaug_hbm.at[src_v.at[j]], gbuf, sem).wait()
        pltpu.sync_copy(gbuf, acc_sh.at[dst_v.at[j]], add=True)
        return carry

    lax.fori_loop(0, CHUNKS_PER_TILE, body, 0)
    plsc.subcore_barrier()
    pltpu.sync_copy(acc_sh.at[pl.ds(s * ROWS_PER_TILE, ROWS_PER_TILE)],
                    out_hbm.at[c].at[pl.ds(s * ROWS_PER_TILE, ROWS_PER_TILE)])


_sc_phase1 = functools.partial(
    pl.kernel,
    out_type=jax.ShapeDtypeStruct((2, N_PAD, 16), jnp.float32),
    mesh=_sc_mesh,
    compiler_params=pltpu.CompilerParams(use_tc_tiling_on_sc=False),
    scratch_types=[
        pltpu.VMEM((CHUNKS_PER_TILE, CHUNK), jnp.int32),
        pltpu.VMEM((CHUNKS_PER_TILE, CHUNK), jnp.int32),
        pltpu.VMEM((CHUNK, 16), jnp.float32),
        pltpu.VMEM_SHARED((N_PAD, 16), jnp.float32),
        pltpu.SemaphoreType.DMA,
    ],
)(_sc1_body)


def _sc2_body(h16_hbm, srcg_hbm, dst_hbm, zeros_hbm, out_hbm,
              src_v, dst_v, gbuf, acc_sh, sem):
    c = lax.axis_index("c")
    s = lax.axis_index("s")
    row0 = s * ROWS_PER_TILE
    pltpu.sync_copy(dst_hbm.at[pl.ds(s * CHUNKS_PER_TILE2, CHUNKS_PER_TILE2)],
                    dst_v)

    def group_body(gi, gcarry):
        g = c * (NG // 2) + gi
        pltpu.sync_copy(
            srcg_hbm.at[pl.ds(g * CHUNKS_TOTAL + s * CHUNKS_PER_TILE2,
                              CHUNKS_PER_TILE2)],
            src_v)
        pltpu.sync_copy(zeros_hbm, acc_sh.at[pl.ds(row0, ROWS_PER_TILE)])
        plsc.subcore_barrier()

        def body(j, carry):
            pltpu.async_copy(h16_hbm.at[src_v.at[j]], gbuf, sem).wait()
            pltpu.sync_copy(gbuf, acc_sh.at[dst_v.at[j]], add=True)
            return carry

        lax.fori_loop(0, CHUNKS_PER_TILE2, body, 0)
        plsc.subcore_barrier()
        pltpu.sync_copy(acc_sh.at[pl.ds(row0, ROWS_PER_TILE)],
                        out_hbm.at[pl.ds(g * N_PAD + row0, ROWS_PER_TILE)])
        plsc.subcore_barrier()
        return gcarry

    lax.fori_loop(0, NG // 2, group_body, 0)


_sc_phase2 = functools.partial(
    pl.kernel,
    out_type=jax.ShapeDtypeStruct((NG * N_PAD, GW), jnp.float32),
    mesh=_sc_mesh,
    compiler_params=pltpu.CompilerParams(use_tc_tiling_on_sc=False),
    scratch_types=[
        pltpu.VMEM((CHUNKS_PER_TILE2, CHUNK), jnp.int32),
        pltpu.VMEM((CHUNKS_PER_TILE2, CHUNK), jnp.int32),
        pltpu.VMEM((CHUNK, GW), jnp.float32),
        pltpu.VMEM_SHARED((N_PAD, GW), jnp.float32),
        pltpu.SemaphoreType.DMA,
    ],
)(_sc2_body)


def _hb_body(x_ref, p0_ref, p1_ref, w0l_ref, b0l_ref, w0r_ref,
             h16_ref, h_ref, invd_ref):
    p0 = p0_ref[...]
    p1 = p1_ref[...]
    deg = p0[:, 0:1] + p1[:, 0:1]
    s1 = p0[:, 1:2] + p1[:, 1:2]
    invd = 1.0 / jnp.maximum(deg, 1.0)
    a = s1 * invd
    x = x_ref[...]
    pre = a * w0l_ref[...] + x * w0r_ref[...] + b0l_ref[...]
    h = jnp.maximum(pre, 0.0) + x
    invd_ref[...] = invd
    h_ref[...] = h
    for g in range(NG):
        h16_ref[g, :, :] = h[:, g * GW:(g + 1) * GW]


def _make_h(x_p, part, W0l, b0l, W0r):
    grid = (N_PAD // BN,)
    return pl.pallas_call(
        _hb_body,
        grid=grid,
        in_specs=[
            pl.BlockSpec((BN, 1), lambda i: (i, 0)),
            pl.BlockSpec((BN, 16), lambda i: (i, 0)),
            pl.BlockSpec((BN, 16), lambda i: (i, 0)),
            pl.BlockSpec((1, 128), lambda i: (0, 0)),
            pl.BlockSpec((1, 128), lambda i: (0, 0)),
            pl.BlockSpec((1, 128), lambda i: (0, 0)),
        ],
        out_specs=[
            pl.BlockSpec((NG, BN, GW), lambda i: (0, i, 0)),
            pl.BlockSpec((BN, 128), lambda i: (i, 0)),
            pl.BlockSpec((BN, 1), lambda i: (i, 0)),
        ],
        out_shape=[
            jax.ShapeDtypeStruct((NG, N_PAD, GW), jnp.float32),
            jax.ShapeDtypeStruct((N_PAD, 128), jnp.float32),
            jax.ShapeDtypeStruct((N_PAD, 1), jnp.float32),
        ],
    )(x_p, part[0], part[1], W0l, b0l.reshape(1, 128), W0r)


def _tail_body(agg2_ref, invd_ref, h_ref, batch_ref,
               w1l_ref, b1l_ref, w1r_ref,
               fc1w_ref, fc1b_ref, fc2w_ref, fc2b_ref,
               out_ref, psum_ref, pcnt_ref):
    i = pl.program_id(0)
    nb = pl.num_programs(0)

    @pl.when(i == 0)
    def _init():
        psum_ref[...] = jnp.zeros_like(psum_ref)
        pcnt_ref[...] = jnp.zeros_like(pcnt_ref)

    h = h_ref[...]
    aggd = agg2_ref[...] * invd_ref[...]
    pre = (jax.lax.dot(aggd, w1l_ref[...], preferred_element_type=jnp.float32)
           + b1l_ref[...]
           + jax.lax.dot(h, w1r_ref[...], preferred_element_type=jnp.float32))
    h2 = jnp.maximum(pre, 0.0) + h

    b = batch_ref[...]
    gids = jax.lax.broadcasted_iota(jnp.int32, (1, 64), 1)
    onehot = jnp.where(b == gids, 1.0, 0.0)
    psum_ref[...] += jax.lax.dot_general(
        onehot, h2, (((0,), (0,)), ((), ())),
        preferred_element_type=jnp.float32,
        precision=jax.lax.Precision.HIGHEST)
    pcnt_ref[...] += jax.lax.dot_general(
        onehot, jnp.ones_like(h2), (((0,), (0,)), ((), ())),
        preferred_element_type=jnp.float32,
        precision=jax.lax.Precision.HIGHEST)

    @pl.when(i == nb - 1)
    def _fin():
        pooled = psum_ref[...] / jnp.maximum(pcnt_ref[...], 1.0)
        z = jnp.maximum(
            jax.lax.dot(pooled, fc1w_ref[...],
                        preferred_element_type=jnp.float32) + fc1b_ref[...],
            0.0)
        out_ref[...] = (jax.lax.dot(z, fc2w_ref[...],
                                    preferred_element_type=jnp.float32)
                        + fc2b_ref[...])


def _tail(agg2_p, invd_p, h_p, batch_p, W1l, b1l, W1r, fc1_W, fc1_b,
          fc2_W, fc2_b):
    grid = (N_PAD // BN,)
    return pl.pallas_call(
        _tail_body,
        grid=grid,
        in_specs=[
            pl.BlockSpec((BN, 128), lambda i: (i, 0)),
            pl.BlockSpec((BN, 1), lambda i: (i, 0)),
            pl.BlockSpec((BN, 128), lambda i: (i, 0)),
            pl.BlockSpec((BN, 1), lambda i: (i, 0)),
            pl.BlockSpec((128, 128), lambda i: (0, 0)),
            pl.BlockSpec((1, 128), lambda i: (0, 0)),
            pl.BlockSpec((128, 128), lambda i: (0, 0)),
            pl.BlockSpec((128, 64), lambda i: (0, 0)),
            pl.BlockSpec((1, 64), lambda i: (0, 0)),
            pl.BlockSpec((64, 1), lambda i: (0, 0)),
            pl.BlockSpec((1, 1), lambda i: (0, 0)),
        ],
        out_specs=pl.BlockSpec((64, 1), lambda i: (0, 0)),
        out_shape=jax.ShapeDtypeStruct((64, 1), jnp.float32),
        scratch_shapes=[
            pltpu.VMEM((64, 128), jnp.float32),
            pltpu.VMEM((64, 128), jnp.float32),
        ],
    )(agg2_p, invd_p, h_p, batch_p, W1l, b1l.reshape(1, 128), W1r,
      fc1_W, fc1_b.reshape(1, 64), fc2_W, fc2_b.reshape(1, 1))


def kernel(x, edge_index, batch, W0l, b0l, W0r, W1l, b1l, W1r,
           fc1_W, fc1_b, fc2_W, fc2_b):
    n = x.shape[0]
    src = edge_index[0]
    dst = edge_index[1]
    epad = E_PAD - src.shape[0]
    src_p = jnp.pad(src, (0, epad))
    dst_p = jnp.pad(dst, (0, epad), constant_values=N)
    dst2d = dst_p.reshape(CHUNKS_TOTAL, CHUNK)
    src2d = src_p.reshape(CHUNKS_TOTAL, CHUNK)
    xaug = jnp.concatenate(
        [jnp.ones((n, 1), jnp.float32), x, jnp.zeros((n, 14), jnp.float32)],
        axis=1)
    zeros16 = jnp.zeros((ROWS_PER_TILE, 16), jnp.float32)
    zeros8 = jnp.zeros((ROWS_PER_TILE, GW), jnp.float32)

    part = _sc_phase1(xaug, src2d, dst2d, zeros16)

    pad = N_PAD - n
    x_p = jnp.pad(x, ((0, pad), (0, 0)))
    batch_p = jnp.pad(batch.reshape(n, 1), ((0, pad), (0, 0)),
                      constant_values=64)

    h16, h_p, invd_p = _make_h(x_p, part, W0l, b0l, W0r)

    srcg = (src2d[None, :, :]
            + (jnp.arange(NG, dtype=jnp.int32) * N_PAD)[:, None, None]
            ).reshape(NG * CHUNKS_TOTAL, CHUNK)
    agg = _sc_phase2(h16.reshape(NG * N_PAD, GW), srcg, dst2d, zeros8)
    agg2_p = (agg.reshape(NG, N_PAD, GW)
              .transpose(1, 0, 2)
              .reshape(N_PAD, 128))

    return _tail(agg2_p, invd_p, h_p, batch_p, W1l, b1l, W1r,
                 fc1_W, fc1_b, fc2_W, fc2_b)


# speedup vs baseline: 4.8018x; 1.7648x over previous
import functools

import jax
import jax.numpy as jnp
from jax import lax
from jax.experimental import pallas as pl
from jax.experimental.pallas import tpu as pltpu
from jax.experimental.pallas import tpu_sc as plsc

N = 50000
N_PAD = 50176
BN = 1024
E = 800000
E_PAD = 819200
CHUNK = 128
NG = 16
GW = 128 // NG
CHUNKS_TOTAL = E_PAD // CHUNK
CHUNKS_PER_TILE = CHUNKS_TOTAL // 32
CHUNKS_PER_TILE2 = CHUNKS_TOTAL // 16
ROWS_PER_TILE = N_PAD // 16

_sc_mesh = plsc.VectorSubcoreMesh(core_axis_name="c", subcore_axis_name="s")

KD = 8


def _pipelined_gather_scatter(table_hbm, srcg_hbm, dstg_hbm, base,
                              src_v, dst_v, acc_sh, gbuf, gsem, ssem,
                              nchunks):
    nsteps = nchunks // KD

    def load_idx(t, p):
        pltpu.sync_copy(srcg_hbm.at[pl.ds(base + t * KD, KD)], src_v.at[p])
        pltpu.sync_copy(dstg_hbm.at[pl.ds(base + t * KD, KD)], dst_v.at[p])

    def fire_gathers(p):
        for b in range(KD):
            pltpu.async_copy(table_hbm.at[src_v.at[p, b]], gbuf.at[p, b],
                             gsem)

    def consume(p):
        for b in range(KD):
            pltpu.make_async_copy(table_hbm.at[src_v.at[p, b]],
                                  gbuf.at[p, b], gsem).wait()
            pltpu.async_copy(gbuf.at[p, b], acc_sh.at[dst_v.at[p, b]],
                             ssem, add=True)

    def drain_scatters(p):
        for b in range(KD):
            pltpu.make_async_copy(gbuf.at[p, b], acc_sh.at[dst_v.at[p, b]],
                                  ssem).wait()

    def body(t, carry):
        p = t % 2

        @pl.when(t >= 2)
        def _drain():
            drain_scatters(p)

        @pl.when(t < nsteps)
        def _fire():
            load_idx(t, p)
            fire_gathers(p)

        @pl.when(t >= 1)
        def _consume():
            consume(1 - p)

        return carry

    lax.fori_loop(0, nsteps + 1, body, 0)
    drain_scatters((nsteps - 1) % 2)


def _sc1_body(xaug_hbm, src_hbm, dst_hbm, zeros_hbm, out_hbm,
              src_v, dst_v, gbuf, acc_sh, gsem, ssem):
    c = lax.axis_index("c")
    s = lax.axis_index("s")
    wid = s * 2 + c
    pltpu.sync_copy(zeros_hbm.at[pl.ds(0, ROWS_PER_TILE)],
                    acc_sh.at[pl.ds(s * ROWS_PER_TILE, ROWS_PER_TILE)])
    plsc.subcore_barrier()
    _pipelined_gather_scatter(xaug_hbm, src_hbm, dst_hbm,
                              wid * CHUNKS_PER_TILE,
                              src_v, dst_v, acc_sh, gbuf,
                              gsem, ssem, CHUNKS_PER_TILE)
    plsc.subcore_barrier()
    pltpu.sync_copy(acc_sh.at[pl.ds(s * ROWS_PER_TILE, ROWS_PER_TILE)],
                    out_hbm.at[c].at[pl.ds(s * ROWS_PER_TILE, ROWS_PER_TILE)])


_sc_phase1 = functools.partial(
    pl.kernel,
    out_type=jax.ShapeDtypeStruct((2, N_PAD, 4), jnp.float32),
    mesh=_sc_mesh,
    compiler_params=pltpu.CompilerParams(use_tc_tiling_on_sc=False),
    scratch_types=[
        pltpu.VMEM((2, KD, CHUNK), jnp.int32),
        pltpu.VMEM((2, KD, CHUNK), jnp.int32),
        pltpu.VMEM((2, KD, CHUNK, 4), jnp.float32),
        pltpu.VMEM_SHARED((N_PAD, 4), jnp.float32),
        pltpu.SemaphoreType.DMA,
        pltpu.SemaphoreType.DMA,
    ],
)(_sc1_body)


def _sc2_body(h16_hbm, srcg_hbm, dst_hbm, zeros_hbm, out_hbm,
              src_v, dst_v, gbuf, acc_sh, gsem, ssem):
    c = lax.axis_index("c")
    s = lax.axis_index("s")
    row0 = s * ROWS_PER_TILE

    def group_body(gi, gcarry):
        g = c * (NG // 2) + gi
        pltpu.sync_copy(zeros_hbm, acc_sh.at[pl.ds(row0, ROWS_PER_TILE)])
        plsc.subcore_barrier()
        _pipelined_gather_scatter(h16_hbm, srcg_hbm, dst_hbm,
                                  g * CHUNKS_TOTAL + s * CHUNKS_PER_TILE2,
                                  src_v, dst_v, acc_sh, gbuf,
                                  gsem, ssem, CHUNKS_PER_TILE2)
        plsc.subcore_barrier()
        pltpu.sync_copy(acc_sh.at[pl.ds(row0, ROWS_PER_TILE)],
                        out_hbm.at[pl.ds(row0, ROWS_PER_TILE),
                                   pl.ds(g * GW, GW)])
        plsc.subcore_barrier()
        return gcarry

    lax.fori_loop(0, NG // 2, group_body, 0)


_sc_phase2 = functools.partial(
    pl.kernel,
    out_type=jax.ShapeDtypeStruct((N_PAD, 128), jnp.float32),
    mesh=_sc_mesh,
    compiler_params=pltpu.CompilerParams(use_tc_tiling_on_sc=False),
    scratch_types=[
        pltpu.VMEM((2, KD, CHUNK), jnp.int32),
        pltpu.VMEM((2, KD, CHUNK), jnp.int32),
        pltpu.VMEM((2, KD, CHUNK, GW), jnp.float32),
        pltpu.VMEM_SHARED((N_PAD, GW), jnp.float32),
        pltpu.SemaphoreType.DMA,
        pltpu.SemaphoreType.DMA,
    ],
)(_sc2_body)


def _hb_body(x_ref, p0_ref, p1_ref, w0l_ref, b0l_ref, w0r_ref,
             h16_ref, h_ref, invd_ref):
    p0 = p0_ref[...]
    p1 = p1_ref[...]
    deg = p0[:, 0:1] + p1[:, 0:1]
    s1 = p0[:, 1:2] + p1[:, 1:2]
    invd = 1.0 / jnp.maximum(deg, 1.0)
    a = s1 * invd
    x = x_ref[...]
    pre = a * w0l_ref[...] + x * w0r_ref[...] + b0l_ref[...]
    h = jnp.maximum(pre, 0.0) + x
    invd_ref[...] = invd
    h_ref[...] = h
    for g in range(NG):
        h16_ref[g, :, :] = h[:, g * GW:(g + 1) * GW]


def _make_h(x_p, part, W0l, b0l, W0r):
    grid = (N_PAD // BN,)
    return pl.pallas_call(
        _hb_body,
        grid=grid,
        in_specs=[
            pl.BlockSpec((BN, 1), lambda i: (i, 0)),
            pl.BlockSpec((BN, 4), lambda i: (i, 0)),
            pl.BlockSpec((BN, 4), lambda i: (i, 0)),
            pl.BlockSpec((1, 128), lambda i: (0, 0)),
            pl.BlockSpec((1, 128), lambda i: (0, 0)),
            pl.BlockSpec((1, 128), lambda i: (0, 0)),
        ],
        out_specs=[
            pl.BlockSpec((NG, BN, GW), lambda i: (0, i, 0)),
            pl.BlockSpec((BN, 128), lambda i: (i, 0)),
            pl.BlockSpec((BN, 1), lambda i: (i, 0)),
        ],
        out_shape=[
            jax.ShapeDtypeStruct((NG, N_PAD, GW), jnp.float32),
            jax.ShapeDtypeStruct((N_PAD, 128), jnp.float32),
            jax.ShapeDtypeStruct((N_PAD, 1), jnp.float32),
        ],
    )(x_p, part[0], part[1], W0l, b0l.reshape(1, 128), W0r)


def _tail_body(agg2_ref, invd_ref, h_ref, batch_ref,
               w1l_ref, b1l_ref, w1r_ref,
               fc1w_ref, fc1b_ref, fc2w_ref, fc2b_ref,
               out_ref, psum_ref, pcnt_ref):
    i = pl.program_id(0)
    nb = pl.num_programs(0)

    @pl.when(i == 0)
    def _init():
        psum_ref[...] = jnp.zeros_like(psum_ref)
        pcnt_ref[...] = jnp.zeros_like(pcnt_ref)

    h = h_ref[...]
    aggd = agg2_ref[...] * invd_ref[...]
    pre = (jax.lax.dot(aggd, w1l_ref[...], preferred_element_type=jnp.float32)
           + b1l_ref[...]
           + jax.lax.dot(h, w1r_ref[...], preferred_element_type=jnp.float32))
    h2 = jnp.maximum(pre, 0.0) + h

    b = batch_ref[...]
    gids = jax.lax.broadcasted_iota(jnp.int32, (1, 64), 1)
    onehot = jnp.where(b == gids, 1.0, 0.0)
    psum_ref[...] += jax.lax.dot_general(
        onehot, h2, (((0,), (0,)), ((), ())),
        preferred_element_type=jnp.float32,
        precision=jax.lax.Precision.HIGHEST)
    pcnt_ref[...] += jax.lax.dot_general(
        onehot, jnp.ones_like(h2), (((0,), (0,)), ((), ())),
        preferred_element_type=jnp.float32,
        precision=jax.lax.Precision.HIGHEST)

    @pl.when(i == nb - 1)
    def _fin():
        pooled = psum_ref[...] / jnp.maximum(pcnt_ref[...], 1.0)
        z = jnp.maximum(
            jax.lax.dot(pooled, fc1w_ref[...],
                        preferred_element_type=jnp.float32) + fc1b_ref[...],
            0.0)
        out_ref[...] = (jax.lax.dot(z, fc2w_ref[...],
                                    preferred_element_type=jnp.float32)
                        + fc2b_ref[...])


def _tail(agg2_p, invd_p, h_p, batch_p, W1l, b1l, W1r, fc1_W, fc1_b,
          fc2_W, fc2_b):
    grid = (N_PAD // BN,)
    return pl.pallas_call(
        _tail_body,
        grid=grid,
        in_specs=[
            pl.BlockSpec((BN, 128), lambda i: (i, 0)),
            pl.BlockSpec((BN, 1), lambda i: (i, 0)),
            pl.BlockSpec((BN, 128), lambda i: (i, 0)),
            pl.BlockSpec((BN, 1), lambda i: (i, 0)),
            pl.BlockSpec((128, 128), lambda i: (0, 0)),
            pl.BlockSpec((1, 128), lambda i: (0, 0)),
            pl.BlockSpec((128, 128), lambda i: (0, 0)),
            pl.BlockSpec((128, 64), lambda i: (0, 0)),
            pl.BlockSpec((1, 64), lambda i: (0, 0)),
            pl.BlockSpec((64, 1), lambda i: (0, 0)),
            pl.BlockSpec((1, 1), lambda i: (0, 0)),
        ],
        out_specs=pl.BlockSpec((64, 1), lambda i: (0, 0)),
        out_shape=jax.ShapeDtypeStruct((64, 1), jnp.float32),
        scratch_shapes=[
            pltpu.VMEM((64, 128), jnp.float32),
            pltpu.VMEM((64, 128), jnp.float32),
        ],
    )(agg2_p, invd_p, h_p, batch_p, W1l, b1l.reshape(1, 128), W1r,
      fc1_W, fc1_b.reshape(1, 64), fc2_W, fc2_b.reshape(1, 1))


def kernel(x, edge_index, batch, W0l, b0l, W0r, W1l, b1l, W1r,
           fc1_W, fc1_b, fc2_W, fc2_b):
    n = x.shape[0]
    src = edge_index[0]
    dst = edge_index[1]
    epad = E_PAD - src.shape[0]
    src_p = jnp.pad(src, (0, epad))
    dst_p = jnp.pad(dst, (0, epad), constant_values=N)
    dst2d = dst_p.reshape(CHUNKS_TOTAL, CHUNK)
    src2d = src_p.reshape(CHUNKS_TOTAL, CHUNK)
    xaug = jnp.concatenate(
        [jnp.ones((n, 1), jnp.float32), x, jnp.zeros((n, 2), jnp.float32)],
        axis=1)
    zeros8 = jnp.zeros((ROWS_PER_TILE, 8), jnp.float32)
    zeros4 = jnp.zeros((ROWS_PER_TILE, 4), jnp.float32)

    part = _sc_phase1(xaug, src2d, dst2d, zeros4)

    pad = N_PAD - n
    x_p = jnp.pad(x, ((0, pad), (0, 0)))
    batch_p = jnp.pad(batch.reshape(n, 1), ((0, pad), (0, 0)),
                      constant_values=64)

    h16, h_p, invd_p = _make_h(x_p, part, W0l, b0l, W0r)

    srcg = (src2d[None, :, :]
            + (jnp.arange(NG, dtype=jnp.int32) * N_PAD)[:, None, None]
            ).reshape(NG * CHUNKS_TOTAL, CHUNK)
    dstg = jnp.broadcast_to(
        dst2d[None], (NG, CHUNKS_TOTAL, CHUNK)
    ).reshape(NG * CHUNKS_TOTAL, CHUNK)
    agg2_p = _sc_phase2(h16.reshape(NG * N_PAD, GW), srcg, dstg, zeros8)

    return _tail(agg2_p, invd_p, h_p, batch_p, W1l, b1l, W1r,
                 fc1_W, fc1_b, fc2_W, fc2_b)
